# baseline (speedup 1.0000x reference)
import functools

import jax
import jax.numpy as jnp
from jax import lax
from jax.experimental import pallas as pl
from jax.experimental.pallas import tpu as pltpu
from jax.experimental.pallas import tpu_sc as plsc


def _sc_gather(u2, i2, c4, W_u2, W_i2, W_c4):
    info = plsc.get_sparse_core_info()
    NC, NS = info.num_cores, info.num_subcores
    NW = NC * NS
    B = u2.shape[0]
    assert B % NW == 0
    b_per_w = B // NW
    mesh = plsc.VectorSubcoreMesh(core_axis_name="c", subcore_axis_name="s")

    @functools.partial(
        pl.kernel,
        mesh=mesh,
        out_type=(
            jax.ShapeDtypeStruct((B, 128), jnp.float32),
            jax.ShapeDtypeStruct((B, 128), jnp.float32),
            jax.ShapeDtypeStruct((B, 128), jnp.float32),
        ),
        scratch_types=[
            pltpu.VMEM((b_per_w,), jnp.int32),
            pltpu.VMEM((b_per_w,), jnp.int32),
            pltpu.VMEM((b_per_w,), jnp.int32),
            pltpu.VMEM((b_per_w, 128), jnp.float32),
            pltpu.VMEM((b_per_w, 128), jnp.float32),
            pltpu.VMEM((b_per_w, 128), jnp.float32),
            pltpu.SemaphoreType.DMA,
        ],
    )
    def k(uid_hbm, iid_hbm, cid_hbm, wu_hbm, wi_hbm, wc_hbm,
          gu_hbm, gi_hbm, gc_hbm,
          uidx, iidx, cidx, urows, irows, crows, sem):
        wid = lax.axis_index("s") * NC + lax.axis_index("c")
        row0 = wid * b_per_w
        pltpu.sync_copy(uid_hbm.at[pl.ds(row0, b_per_w)], uidx)
        pltpu.sync_copy(iid_hbm.at[pl.ds(row0, b_per_w)], iidx)
        pltpu.sync_copy(cid_hbm.at[pl.ds(row0, b_per_w)], cidx)
        cu = pltpu.async_copy(wu_hbm.at[uidx], urows, sem)
        ci = pltpu.async_copy(wi_hbm.at[iidx], irows, sem)
        cc = pltpu.async_copy(wc_hbm.at[cidx], crows, sem)
        cu.wait()
        ci.wait()
        cc.wait()
        pltpu.sync_copy(urows, gu_hbm.at[pl.ds(row0, b_per_w)])
        pltpu.sync_copy(irows, gi_hbm.at[pl.ds(row0, b_per_w)])
        pltpu.sync_copy(crows, gc_hbm.at[pl.ds(row0, b_per_w)])

    return k(u2, i2, c4, W_u2, W_i2, W_c4)


def _tc_project(base, g_u, g_i, g_c, up, ip, cp, W0, Wu2, Wi2, Wc4, b_proj):
    B, DB = base.shape
    N = W0.shape[1]
    BLK = 512
    grid = (B // BLK,)

    def body(base_ref, gu_ref, gi_ref, gc_ref, up_ref, ip_ref, cp_ref,
             w0_ref, wu_ref, wi_ref, wc_ref, b_ref, out_ref):
        lane = lax.broadcasted_iota(jnp.int32, (BLK, 128), 1)
        half = lane // 64
        quarter = lane // 32
        mu = (half == up_ref[...]).astype(jnp.float32)
        mi = (half == ip_ref[...]).astype(jnp.float32)
        mc = (quarter == cp_ref[...]).astype(jnp.float32)
        acc = jnp.dot(base_ref[...], w0_ref[...],
                      preferred_element_type=jnp.float32)
        acc += jnp.dot(gu_ref[...] * mu, wu_ref[...],
                       preferred_element_type=jnp.float32)
        acc += jnp.dot(gi_ref[...] * mi, wi_ref[...],
                       preferred_element_type=jnp.float32)
        acc += jnp.dot(gc_ref[...] * mc, wc_ref[...],
                       preferred_element_type=jnp.float32)
        out_ref[...] = acc + b_ref[...]

    return pl.pallas_call(
        body,
        grid=grid,
        in_specs=[
            pl.BlockSpec((BLK, DB), lambda i: (i, 0)),
            pl.BlockSpec((BLK, 128), lambda i: (i, 0)),
            pl.BlockSpec((BLK, 128), lambda i: (i, 0)),
            pl.BlockSpec((BLK, 128), lambda i: (i, 0)),
            pl.BlockSpec((BLK, 1), lambda i: (i, 0)),
            pl.BlockSpec((BLK, 1), lambda i: (i, 0)),
            pl.BlockSpec((BLK, 1), lambda i: (i, 0)),
            pl.BlockSpec((DB, N), lambda i: (0, 0)),
            pl.BlockSpec((128, N), lambda i: (0, 0)),
            pl.BlockSpec((128, N), lambda i: (0, 0)),
            pl.BlockSpec((128, N), lambda i: (0, 0)),
            pl.BlockSpec((1, N), lambda i: (0, 0)),
        ],
        out_specs=pl.BlockSpec((BLK, N), lambda i: (i, 0)),
        out_shape=jax.ShapeDtypeStruct((B, N), jnp.float32),
    )(base, g_u, g_i, g_c, up, ip, cp, W0, Wu2, Wi2, Wc4, b_proj.reshape(1, N))


def kernel(base, user_id, item_id, category, W_user, W_item, W_cat, W_proj, b_proj):
    B = base.shape[0]
    user_id = user_id.astype(jnp.int32)
    item_id = item_id.astype(jnp.int32)
    category = category.astype(jnp.int32)

    W_u2 = W_user.reshape(W_user.shape[0] // 2, 128)
    W_i2 = W_item.reshape(W_item.shape[0] // 2, 128)
    W_c4 = W_cat.reshape(W_cat.shape[0] // 4, 128)
    u2, up = user_id // 2, (user_id % 2).reshape(B, 1)
    i2, ip = item_id // 2, (item_id % 2).reshape(B, 1)
    c4, cp = category // 4, (category % 4).reshape(B, 1)

    g_u, g_i, g_c = _sc_gather(u2, i2, c4, W_u2, W_i2, W_c4)

    Wu2p = jnp.concatenate([W_proj[128:192], W_proj[128:192]], axis=0)
    Wi2p = jnp.concatenate([W_proj[192:256], W_proj[192:256]], axis=0)
    Wc4p = jnp.concatenate([W_proj[256:288]] * 4, axis=0)
    return _tc_project(base, g_u, g_i, g_c, up, ip, cp,
                       W_proj[0:128], Wu2p, Wi2p, Wc4p, b_proj)


# speedup vs baseline: 2.5121x; 2.5121x over previous
import functools

import jax
import jax.numpy as jnp
from jax import lax
from jax.experimental import pallas as pl
from jax.experimental.pallas import tpu as pltpu
from jax.experimental.pallas import tpu_sc as plsc


def _sc_gather_t(user_id, item_id, category, wuT, wiT, wcT):
    info = plsc.get_sparse_core_info()
    NC, NS = info.num_cores, info.num_subcores
    NW = NC * NS
    B = user_id.shape[0]
    DU = wuT.shape[0]
    VU = wuT.shape[1]
    DC = wcT.shape[0]
    VC = wcT.shape[1]
    assert DU == 2 * NW and DC == NW
    n_iter = B // 16
    mesh = plsc.VectorSubcoreMesh(core_axis_name="c", subcore_axis_name="s")

    @functools.partial(
        pl.kernel,
        mesh=mesh,
        compiler_params=pltpu.CompilerParams(needs_layout_passes=False),
        out_type=(
            jax.ShapeDtypeStruct((DU, B), jnp.float32),
            jax.ShapeDtypeStruct((DU, B), jnp.float32),
            jax.ShapeDtypeStruct((DC, B), jnp.float32),
        ),
        scratch_types=[
            pltpu.VMEM((B,), jnp.int32),
            pltpu.VMEM((B,), jnp.int32),
            pltpu.VMEM((B,), jnp.int32),
            pltpu.VMEM((VU,), jnp.float32),
            pltpu.VMEM((VC,), jnp.float32),
            pltpu.VMEM((B,), jnp.float32),
            pltpu.SemaphoreType.DMA,
        ],
    )
    def k(uid_h, iid_h, cid_h, wu_h, wi_h, wc_h, eu_h, ei_h, ec_h,
          uidx, iidx, cidx, rowbuf, catbuf, outbuf, sem):
        w = lax.axis_index("s") * NC + lax.axis_index("c")
        pltpu.sync_copy(uid_h, uidx)
        pltpu.sync_copy(iid_h, iidx)
        pltpu.sync_copy(cid_h, cidx)

        def run_task(tbl_h, row, idxbuf, out_h, buf):
            pltpu.sync_copy(tbl_h.at[row], buf)

            def body(i, carry):
                iv = idxbuf[pl.ds(i * 16, 16)]
                outbuf[pl.ds(i * 16, 16)] = plsc.load_gather(buf, [iv])
                return carry

            lax.fori_loop(0, n_iter, body, 0)
            pltpu.sync_copy(outbuf, out_h.at[row])

        run_task(wu_h, w, uidx, eu_h, rowbuf)
        run_task(wu_h, w + NW, uidx, eu_h, rowbuf)
        run_task(wi_h, w, iidx, ei_h, rowbuf)
        run_task(wi_h, w + NW, iidx, ei_h, rowbuf)
        run_task(wc_h, w, cidx, ec_h, catbuf)

    return k(user_id, item_id, category, wuT, wiT, wcT)


def _tc_project(base, euT, eiT, ecT, W_proj, b_proj):
    B, DB = base.shape
    DU = euT.shape[0]
    DI = eiT.shape[0]
    DC = ecT.shape[0]
    N = W_proj.shape[1]
    K = W_proj.shape[0]
    BLK = 512
    grid = (B // BLK,)
    dn_t = (((0,), (0,)), ((), ()))

    def body(base_ref, eu_ref, ei_ref, ec_ref, wp_ref, b_ref, out_ref):
        acc = jnp.dot(base_ref[...], wp_ref[0:DB, :],
                      preferred_element_type=jnp.float32)
        acc += lax.dot_general(eu_ref[...], wp_ref[DB:DB + DU, :], dn_t,
                               preferred_element_type=jnp.float32)
        acc += lax.dot_general(ei_ref[...], wp_ref[DB + DU:DB + DU + DI, :],
                               dn_t, preferred_element_type=jnp.float32)
        acc += lax.dot_general(ec_ref[...], wp_ref[DB + DU + DI:K, :], dn_t,
                               preferred_element_type=jnp.float32)
        out_ref[...] = acc + b_ref[...]

    return pl.pallas_call(
        body,
        grid=grid,
        in_specs=[
            pl.BlockSpec((BLK, DB), lambda i: (i, 0)),
            pl.BlockSpec((DU, BLK), lambda i: (0, i)),
            pl.BlockSpec((DI, BLK), lambda i: (0, i)),
            pl.BlockSpec((DC, BLK), lambda i: (0, i)),
            pl.BlockSpec((K, N), lambda i: (0, 0)),
            pl.BlockSpec((1, N), lambda i: (0, 0)),
        ],
        out_specs=pl.BlockSpec((BLK, N), lambda i: (i, 0)),
        out_shape=jax.ShapeDtypeStruct((B, N), jnp.float32),
    )(base, euT, eiT, ecT, W_proj, b_proj.reshape(1, N))


def kernel(base, user_id, item_id, category, W_user, W_item, W_cat, W_proj, b_proj):
    user_id = user_id.astype(jnp.int32)
    item_id = item_id.astype(jnp.int32)
    category = category.astype(jnp.int32)
    euT, eiT, ecT = _sc_gather_t(user_id, item_id, category,
                                 W_user.T, W_item.T, W_cat.T)
    return _tc_project(base, euT, eiT, ecT, W_proj, b_proj)


# speedup vs baseline: 2.8172x; 1.1215x over previous
import functools

import jax
import jax.numpy as jnp
from jax import lax
from jax.experimental import pallas as pl
from jax.experimental.pallas import tpu as pltpu
from jax.experimental.pallas import tpu_sc as plsc


def _sc_gather_t(user_id, item_id, category, wuT, wiT, wcT):
    info = plsc.get_sparse_core_info()
    NC, NS = info.num_cores, info.num_subcores
    NW = NC * NS
    B = user_id.shape[0]
    DU, VU = wuT.shape
    DC, VC = wcT.shape
    assert DU == 2 * NW and DC == NW
    UNROLL = 4
    n_iter = B // (16 * UNROLL)
    mesh = plsc.VectorSubcoreMesh(core_axis_name="c", subcore_axis_name="s")

    @functools.partial(
        pl.kernel,
        mesh=mesh,
        compiler_params=pltpu.CompilerParams(needs_layout_passes=False),
        out_type=(
            jax.ShapeDtypeStruct((DU, B), jnp.float32),
            jax.ShapeDtypeStruct((DU, B), jnp.float32),
            jax.ShapeDtypeStruct((DC, B), jnp.float32),
        ),
        scratch_types=[
            pltpu.VMEM((B,), jnp.int32),
            pltpu.VMEM((B,), jnp.int32),
            pltpu.VMEM((B,), jnp.int32),
            pltpu.VMEM((VU,), jnp.float32),
            pltpu.VMEM((VC,), jnp.float32),
            pltpu.VMEM((B,), jnp.float32),
            pltpu.VMEM((B,), jnp.float32),
            pltpu.SemaphoreType.DMA,
            pltpu.SemaphoreType.DMA,
            pltpu.SemaphoreType.DMA,
        ],
    )
    def k(uid_h, iid_h, cid_h, wu_h, wi_h, wc_h, eu_h, ei_h, ec_h,
          uidx, iidx, cidx, rowbuf, catbuf, outA, outB,
          sem_row, sem_out, sem_pre):
        w = lax.axis_index("s") * NC + lax.axis_index("c")
        tasks = [(wu_h, w, uidx, eu_h), (wu_h, w + NW, uidx, eu_h),
                 (wi_h, w, iidx, ei_h), (wi_h, w + NW, iidx, ei_h)]
        outs = [outA, outB]

        row_copy = pltpu.async_copy(wu_h.at[w], rowbuf, sem_row)
        pre = [pltpu.async_copy(uid_h, uidx, sem_pre),
               pltpu.async_copy(iid_h, iidx, sem_pre),
               pltpu.async_copy(cid_h, cidx, sem_pre),
               pltpu.async_copy(wc_h.at[w], catbuf, sem_pre)]
        for c in pre:
            c.wait()

        def gather(idxbuf, buf, ob):
            def body(i, carry):
                for u in range(UNROLL):
                    off = (i * UNROLL + u) * 16
                    iv = idxbuf[pl.ds(off, 16)]
                    ob[pl.ds(off, 16)] = plsc.load_gather(buf, [iv])
                return carry

            lax.fori_loop(0, n_iter, body, 0)

        out_copies = {}
        for t in range(4):
            row_copy.wait()
            ob = outs[t % 2]
            if t >= 2:
                out_copies.pop(t - 2).wait()
            gather(tasks[t][2], rowbuf, ob)
            if t < 3:
                tbl2, row2, _, _ = tasks[t + 1]
                row_copy = pltpu.async_copy(tbl2.at[row2], rowbuf, sem_row)
            out_copies[t] = pltpu.async_copy(
                ob, tasks[t][3].at[tasks[t][1]], sem_out)

        out_copies.pop(2).wait()
        gather(cidx, catbuf, outA)
        cw = pltpu.async_copy(outA, ec_h.at[w], sem_out)
        out_copies.pop(3).wait()
        cw.wait()

    return k(user_id, item_id, category, wuT, wiT, wcT)


def _tc_project(base, euT, eiT, ecT, W_proj, b_proj):
    B, DB = base.shape
    DU = euT.shape[0]
    DI = eiT.shape[0]
    DC = ecT.shape[0]
    N = W_proj.shape[1]
    K = W_proj.shape[0]
    BLK = 1024
    grid = (B // BLK,)
    dn_t = (((0,), (0,)), ((), ()))

    def body(base_ref, eu_ref, ei_ref, ec_ref, wp_ref, b_ref, out_ref):
        acc = jnp.dot(base_ref[...], wp_ref[0:DB, :],
                      preferred_element_type=jnp.float32)
        acc += lax.dot_general(eu_ref[...], wp_ref[DB:DB + DU, :], dn_t,
                               preferred_element_type=jnp.float32)
        acc += lax.dot_general(ei_ref[...], wp_ref[DB + DU:DB + DU + DI, :],
                               dn_t, preferred_element_type=jnp.float32)
        acc += lax.dot_general(ec_ref[...], wp_ref[DB + DU + DI:K, :], dn_t,
                               preferred_element_type=jnp.float32)
        out_ref[...] = acc + b_ref[...]

    return pl.pallas_call(
        body,
        grid=grid,
        in_specs=[
            pl.BlockSpec((BLK, DB), lambda i: (i, 0)),
            pl.BlockSpec((DU, BLK), lambda i: (0, i)),
            pl.BlockSpec((DI, BLK), lambda i: (0, i)),
            pl.BlockSpec((DC, BLK), lambda i: (0, i)),
            pl.BlockSpec((K, N), lambda i: (0, 0)),
            pl.BlockSpec((1, N), lambda i: (0, 0)),
        ],
        out_specs=pl.BlockSpec((BLK, N), lambda i: (i, 0)),
        out_shape=jax.ShapeDtypeStruct((B, N), jnp.float32),
    )(base, euT, eiT, ecT, W_proj, b_proj.reshape(1, N))


def kernel(base, user_id, item_id, category, W_user, W_item, W_cat, W_proj, b_proj):
    user_id = user_id.astype(jnp.int32)
    item_id = item_id.astype(jnp.int32)
    category = category.astype(jnp.int32)
    euT, eiT, ecT = _sc_gather_t(user_id, item_id, category,
                                 W_user.T, W_item.T, W_cat.T)
    return _tc_project(base, euT, eiT, ecT, W_proj, b_proj)


# speedup vs baseline: 2.8866x; 1.0246x over previous
import functools

import jax
import jax.numpy as jnp
from jax import lax
from jax.experimental import pallas as pl
from jax.experimental.pallas import tpu as pltpu
from jax.experimental.pallas import tpu_sc as plsc


def _sc_gather_t(user_id, item_id, category, wuT, wiT, wcT):
    info = plsc.get_sparse_core_info()
    NC, NS = info.num_cores, info.num_subcores
    NW = NC * NS
    B = user_id.shape[0]
    DU, VU = wuT.shape
    DC, VC = wcT.shape
    assert DU == 2 * NW and DC == NW
    UNROLL = 4
    n_iter = B // (16 * UNROLL)
    mesh = plsc.VectorSubcoreMesh(core_axis_name="c", subcore_axis_name="s")

    @functools.partial(
        pl.kernel,
        mesh=mesh,
        compiler_params=pltpu.CompilerParams(needs_layout_passes=False),
        out_type=(
            jax.ShapeDtypeStruct((DU, B), jnp.float32),
            jax.ShapeDtypeStruct((DU, B), jnp.float32),
            jax.ShapeDtypeStruct((DC, B), jnp.float32),
        ),
        scratch_types=[
            pltpu.VMEM((B,), jnp.int32),
            pltpu.VMEM((B,), jnp.int32),
            pltpu.VMEM((B,), jnp.int32),
            pltpu.VMEM((VU,), jnp.float32),
            pltpu.VMEM((VC,), jnp.float32),
            pltpu.VMEM((B,), jnp.float32),
            pltpu.VMEM((B,), jnp.float32),
            pltpu.SemaphoreType.DMA,
            pltpu.SemaphoreType.DMA,
            pltpu.SemaphoreType.DMA,
        ],
    )
    def k(uid_h, iid_h, cid_h, wu_h, wi_h, wc_h, eu_h, ei_h, ec_h,
          uidx, iidx, cidx, rowbuf, catbuf, outA, outB,
          sem_row, sem_out, sem_pre):
        w = lax.axis_index("s") * NC + lax.axis_index("c")
        tasks = [(wu_h, w, uidx, eu_h), (wu_h, w + NW, uidx, eu_h),
                 (wi_h, w, iidx, ei_h), (wi_h, w + NW, iidx, ei_h)]
        outs = [outA, outB]

        row_copy = pltpu.async_copy(wu_h.at[w], rowbuf, sem_row)
        pre = [pltpu.async_copy(uid_h, uidx, sem_pre),
               pltpu.async_copy(iid_h, iidx, sem_pre),
               pltpu.async_copy(cid_h, cidx, sem_pre),
               pltpu.async_copy(wc_h.at[w], catbuf, sem_pre)]
        for c in pre:
            c.wait()

        def gather(idxbuf, buf, ob):
            def body(i, carry):
                for u in range(UNROLL):
                    off = (i * UNROLL + u) * 16
                    iv = idxbuf[pl.ds(off, 16)]
                    ob[pl.ds(off, 16)] = plsc.load_gather(buf, [iv])
                return carry

            lax.fori_loop(0, n_iter, body, 0)

        out_copies = {}
        for t in range(4):
            row_copy.wait()
            ob = outs[t % 2]
            if t >= 2:
                out_copies.pop(t - 2).wait()
            gather(tasks[t][2], rowbuf, ob)
            if t < 3:
                tbl2, row2, _, _ = tasks[t + 1]
                row_copy = pltpu.async_copy(tbl2.at[row2], rowbuf, sem_row)
            out_copies[t] = pltpu.async_copy(
                ob, tasks[t][3].at[tasks[t][1]], sem_out)

        out_copies.pop(2).wait()
        gather(cidx, catbuf, outA)
        cw = pltpu.async_copy(outA, ec_h.at[w], sem_out)
        out_copies.pop(3).wait()
        cw.wait()

    return k(user_id, item_id, category, wuT, wiT, wcT)


def _tc_project(base, euT, eiT, ecT, W_proj, b_proj):
    B, DB = base.shape
    DU = euT.shape[0]
    DI = eiT.shape[0]
    DC = ecT.shape[0]
    N = W_proj.shape[1]
    K = W_proj.shape[0]
    BLK = 2048
    grid = (B // BLK,)
    dn_t = (((0,), (0,)), ((), ()))

    def body(base_ref, eu_ref, ei_ref, ec_ref, wp_ref, b_ref, out_ref):
        acc = jnp.dot(base_ref[...], wp_ref[0:DB, :],
                      preferred_element_type=jnp.float32)
        acc += lax.dot_general(eu_ref[...], wp_ref[DB:DB + DU, :], dn_t,
                               preferred_element_type=jnp.float32)
        acc += lax.dot_general(ei_ref[...], wp_ref[DB + DU:DB + DU + DI, :],
                               dn_t, preferred_element_type=jnp.float32)
        acc += lax.dot_general(ec_ref[...], wp_ref[DB + DU + DI:K, :], dn_t,
                               preferred_element_type=jnp.float32)
        out_ref[...] = acc + b_ref[...]

    return pl.pallas_call(
        body,
        grid=grid,
        in_specs=[
            pl.BlockSpec((BLK, DB), lambda i: (i, 0)),
            pl.BlockSpec((DU, BLK), lambda i: (0, i)),
            pl.BlockSpec((DI, BLK), lambda i: (0, i)),
            pl.BlockSpec((DC, BLK), lambda i: (0, i)),
            pl.BlockSpec((K, N), lambda i: (0, 0)),
            pl.BlockSpec((1, N), lambda i: (0, 0)),
        ],
        out_specs=pl.BlockSpec((BLK, N), lambda i: (i, 0)),
        out_shape=jax.ShapeDtypeStruct((B, N), jnp.float32),
    )(base, euT, eiT, ecT, W_proj, b_proj.reshape(1, N))


def kernel(base, user_id, item_id, category, W_user, W_item, W_cat, W_proj, b_proj):
    user_id = user_id.astype(jnp.int32)
    item_id = item_id.astype(jnp.int32)
    category = category.astype(jnp.int32)
    euT, eiT, ecT = _sc_gather_t(user_id, item_id, category,
                                 W_user.T, W_item.T, W_cat.T)
    return _tc_project(base, euT, eiT, ecT, W_proj, b_proj)
